# baseline (speedup 1.0000x reference)
import functools

import jax
import jax.numpy as jnp
from jax.experimental import pallas as pl

B = 32768
DS = 658
DB = 250
H = 256
A = 20
BM = 1024
OUT_W = 128


def _body(xs_ref, xb_ref, act_ref, ws_ref, wb_ref, b1_ref, w2_ref, b2_ref,
          lp_ref, val_ref, ent_ref):
    acc = jnp.dot(xs_ref[...], ws_ref[...], preferred_element_type=jnp.float32)
    acc = acc + jnp.dot(xb_ref[...], wb_ref[...],
                        preferred_element_type=jnp.float32)
    h = jnp.tanh(acc + b1_ref[...])
    o2 = jnp.dot(h, w2_ref[...], preferred_element_type=jnp.float32) + b2_ref[...]
    logits = o2[:, :A]
    value = o2[:, A]
    m = jnp.max(logits, axis=-1, keepdims=True)
    e = jnp.exp(logits - m)
    z = jnp.sum(e, axis=-1, keepdims=True)
    logp = logits - m - jnp.log(z)
    ent = -jnp.sum((e / z) * logp, axis=-1)
    onehot = jax.lax.broadcasted_iota(jnp.int32, logits.shape, 1) == act_ref[...]
    alp = jnp.sum(jnp.where(onehot, logp, 0.0), axis=-1)
    lp_ref[...] = alp[:, None]
    val_ref[...] = value[:, None]
    ent_ref[...] = ent[:, None]


@functools.partial(jax.jit, static_argnames=("interpret",))
def _run(states, believes, actions, Wa1, ba1, Wa2, ba2, Wc1, bc1, Wc2, bc2,
         interpret=False):
    ws = jnp.concatenate([Wa1[:DS], Wc1], axis=1)
    wb = jnp.concatenate([Wa1[DS:], jnp.zeros((DB, H), jnp.float32)], axis=1)
    b1 = jnp.concatenate([ba1, bc1])[None, :]
    w2 = jnp.zeros((2 * H, OUT_W), jnp.float32)
    w2 = w2.at[:H, :A].set(Wa2).at[H:, A].set(Wc2[:, 0])
    b2 = jnp.zeros((OUT_W,), jnp.float32).at[:A].set(ba2).at[A].set(bc2[0])
    b2 = b2[None, :]
    act2d = actions.astype(jnp.int32)[:, None]

    grid = (B // BM,)
    out = pl.pallas_call(
        _body,
        grid=grid,
        in_specs=[
            pl.BlockSpec((BM, DS), lambda i: (i, 0)),
            pl.BlockSpec((BM, DB), lambda i: (i, 0)),
            pl.BlockSpec((BM, 1), lambda i: (i, 0)),
            pl.BlockSpec((DS, 2 * H), lambda i: (0, 0)),
            pl.BlockSpec((DB, 2 * H), lambda i: (0, 0)),
            pl.BlockSpec((1, 2 * H), lambda i: (0, 0)),
            pl.BlockSpec((2 * H, OUT_W), lambda i: (0, 0)),
            pl.BlockSpec((1, OUT_W), lambda i: (0, 0)),
        ],
        out_specs=[
            pl.BlockSpec((BM, 1), lambda i: (i, 0)),
            pl.BlockSpec((BM, 1), lambda i: (i, 0)),
            pl.BlockSpec((BM, 1), lambda i: (i, 0)),
        ],
        out_shape=[jax.ShapeDtypeStruct((B, 1), jnp.float32)] * 3,
        interpret=interpret,
    )(states, believes, act2d, ws, wb, b1, w2, b2)
    return out[0][:, 0], out[1][:, 0], out[2][:, 0]


def kernel(states, believes, actions, Wa1, ba1, Wa2, ba2, Wc1, bc1, Wc2, bc2):
    return _run(states, believes, actions, Wa1, ba1, Wa2, ba2,
                Wc1, bc1, Wc2, bc2)
